# baseline (speedup 1.0000x reference)
import dataclasses
import functools

import jax
import jax.numpy as jnp
from jax import lax
from jax.experimental import pallas as pl
from jax.experimental.pallas import tpu as pltpu
from jax.experimental.pallas import tpu_sc as plsc

N = 10000
E = 160000
N_PAD = 10240
N_SP = 10112
TRASH = 10000
E_PAD = 163840
BM = 1024
PREC = jax.lax.Precision.HIGHEST


def _stage_a_body(x_ref, wax_ref, ba_ref, g_ref):
    g_ref[0] = jnp.dot(x_ref[...], wax_ref[...], precision=PREC,
                       preferred_element_type=jnp.float32) + ba_ref[...]


def _stage_a(x, wax, ba):
    fi, h = wax.shape
    nc = h // 128
    nm = N_PAD // BM
    grid = (nm, nc)
    return pl.pallas_call(
        _stage_a_body,
        grid=grid,
        in_specs=[
            pl.BlockSpec((BM, fi), lambda m, o: (m, 0)),
            pl.BlockSpec((fi, 128), lambda m, o: (0, o)),
            pl.BlockSpec((1, 128), lambda m, o: (0, o)),
        ],
        out_specs=pl.BlockSpec((1, BM, 128), lambda m, o: (o, m, 0)),
        out_shape=jax.ShapeDtypeStruct((nc, N_PAD, 128), jnp.float32),
    )(x, wax, ba)


def _stage_c_body(nk, relu, s_ref, wb_ref, cnt_ref, bb_ref, o_ref, acc_ref):
    k = pl.program_id(2)

    @pl.when(k == 0)
    def _():
        acc_ref[...] = jnp.zeros_like(acc_ref)

    acc_ref[...] += jnp.dot(s_ref[0], wb_ref[0], precision=PREC,
                            preferred_element_type=jnp.float32)

    @pl.when(k == nk - 1)
    def _():
        csum = jnp.sum(cnt_ref[...], axis=(0, 2))[:, None]
        rc = 1.0 / jnp.maximum(csum, 1.0)
        ind = jnp.minimum(csum, 1.0)
        out = acc_ref[...] * rc + ind * bb_ref[...]
        if relu:
            out = jnp.maximum(out, 0.0)
        o_ref[...] = out


def _stage_c(s, wb3, cnt2, bb, relu):
    nk = wb3.shape[0]
    o = wb3.shape[2]
    bo = min(o, 256)
    nm = N_PAD // BM
    no = o // bo
    grid = (nm, no, nk)
    return pl.pallas_call(
        functools.partial(_stage_c_body, nk, relu),
        grid=grid,
        in_specs=[
            pl.BlockSpec((1, BM, 128), lambda m, o_, k: (k, m, 0)),
            pl.BlockSpec((1, 128, bo), lambda m, o_, k: (k, 0, o_)),
            pl.BlockSpec((2, BM, 16), lambda m, o_, k: (0, m, 0)),
            pl.BlockSpec((1, bo), lambda m, o_, k: (0, o_)),
        ],
        out_specs=pl.BlockSpec((BM, bo), lambda m, o_, k: (m, o_)),
        out_shape=jax.ShapeDtypeStruct((N_PAD, o), jnp.float32),
        scratch_shapes=[pltpu.VMEM((BM, bo), jnp.float32)],
    )(s, wb3, cnt2, bb)


def _dense_body(relu, x_ref, w_ref, b_ref, o_ref):
    out = jnp.dot(x_ref[...], w_ref[...], precision=PREC,
                  preferred_element_type=jnp.float32) + b_ref[...]
    if relu:
        out = jnp.maximum(out, 0.0)
    o_ref[...] = out


def _dense(x, w, b, relu):
    k, o = w.shape
    bo = min(o, 512)
    grid = (N_PAD // BM, o // bo)
    return pl.pallas_call(
        functools.partial(_dense_body, relu),
        grid=grid,
        in_specs=[
            pl.BlockSpec((BM, k), lambda m, o_: (m, 0)),
            pl.BlockSpec((k, bo), lambda m, o_: (0, o_)),
            pl.BlockSpec((1, bo), lambda m, o_: (0, o_)),
        ],
        out_specs=pl.BlockSpec((BM, bo), lambda m, o_: (m, o_)),
        out_shape=jax.ShapeDtypeStruct((N_PAD, o), jnp.float32),
    )(x, w, b)


STRIPE = N_SP // 16


def _sc_mesh():
    return plsc.VectorSubcoreMesh(core_axis_name="c", subcore_axis_name="s")


def _sc_params():
    cp = pltpu.CompilerParams()
    if "needs_layout_passes" in pltpu.CompilerParams.__dataclass_fields__:
        cp = dataclasses.replace(cp, needs_layout_passes=False)
    return cp


_SYNC = True


def _make_sc_edge(nc):
    edge_split = nc == 1
    passes = 1 if nc <= 2 else nc // 2
    n_out = 2 if nc == 1 else nc
    B = 64
    nb = 80 if edge_split else 160

    out_type = [jax.ShapeDtypeStruct((n_out * N_PAD, 128), jnp.float32)]

    scratch_types = (
        [pltpu.VMEM((B, 128), jnp.float32) for _ in range(2)]
        + [pltpu.VMEM((B,), jnp.int32) for _ in range(4)]
        + [pltpu.VMEM((B,), jnp.float32) for _ in range(4)]
        + [
            pltpu.VMEM((256,), jnp.float32),
            pltpu.VMEM_SHARED((N_SP, 128), jnp.float32),
        ]
        + [pltpu.SemaphoreType.DMA for _ in range(4)]
    )

    def body(g2, src_hbm, dst_hbm, dx_hbm, dy_hbm, wsp_hbm, z128, s_out,
             *scr):
        rows = scr[0:2]
        idd = scr[2:4]
        ids2 = scr[4:6]
        dxb = scr[6:8]
        dyb = scr[8:10]
        w_v, s_sh = scr[10:12]
        gsem = scr[12:14]
        ssem = scr[14:16]

        core = lax.axis_index("c")
        sid = lax.axis_index("s")
        r0 = sid * STRIPE

        egids = [lax.iota(jnp.int32, 16) + 16 * j for j in range(4)]

        if edge_split:
            ebase = core * (E_PAD // 2) + sid * (nb * B)
        else:
            ebase = sid * (nb * B)

        def prefetch(b, i, goff):
            off = ebase + b * B
            pltpu.sync_copy(src_hbm.at[pl.ds(off, B)], ids2[i])
            pltpu.sync_copy(dst_hbm.at[pl.ds(off, B)], idd[i])
            pltpu.sync_copy(dx_hbm.at[pl.ds(off, B)], dxb[i])
            pltpu.sync_copy(dy_hbm.at[pl.ds(off, B)], dyb[i])
            for j in range(4):
                sl = pl.ds(j * 16, 16)
                ids2[i][sl] = ids2[i][sl] + goff
            pltpu.async_copy(g2.at[ids2[i]], rows[i], gsem[i])

        def wait_gather(i):
            pltpu.make_async_copy(g2.at[pl.ds(0, B)], rows[i], gsem[i]).wait()

        def wait_scat(i):
            pltpu.make_async_copy(g2.at[pl.ds(0, B)], rows[i], ssem[i]).wait()

        def compute(i):
            dxs = [dxb[i][pl.ds(j * 16, 16)] for j in range(4)]
            dys = [dyb[i][pl.ds(j * 16, 16)] for j in range(4)]

            @pl.loop(0, 128)
            def _(f):
                fsplat = jnp.full((16,), f, jnp.int32)
                w0f = plsc.load_gather(w_v, [fsplat])
                w1f = plsc.load_gather(w_v, [jnp.full((16,), f + 128,
                                                      jnp.int32)])
                for j in range(4):
                    g = plsc.load_gather(rows[i], [egids[j], fsplat])
                    m = jnp.maximum(g + dxs[j] * w0f + dys[j] * w1f, 0.0)
                    plsc.store_scatter(rows[i], [egids[j], fsplat], m)

        def phase(b, i, goff, first=False, do_prefetch=True):
            wait_gather(i)
            compute(i)
            pltpu.async_copy(rows[i], s_sh.at[idd[i]], ssem[i], add=True)
            if not first:
                wait_scat(1 - i)
            if do_prefetch:
                pf = b + 1
                if isinstance(pf, int):
                    if pf < nb:
                        prefetch(pf, 1 - i, goff)
                else:
                    @pl.when(pf < nb)
                    def _():
                        prefetch(pf, 1 - i, goff)

        for p in range(passes):
            gc = 0 if edge_split else core * passes + p
            goff = jnp.full((16,), gc * N_PAD, jnp.int32)
            pltpu.sync_copy(wsp_hbm.at[pl.ds(gc * 256, 256)], w_v)
            pltpu.sync_copy(z128, s_sh.at[pl.ds(r0, STRIPE)])
            plsc.subcore_barrier()

            if _SYNC:
                @pl.loop(0, nb)
                def _(b):
                    prefetch(b, 0, goff)
                    wait_gather(0)
                    compute(0)
                    pltpu.sync_copy(rows[0], s_sh.at[idd[0]], add=True)
            else:
                prefetch(0, 0, goff)
                phase(0, 0, goff, first=True)
                k = (nb - 2) // 2
                @pl.loop(1, 1 + 2 * k, step=2)
                def _(b):
                    phase(b, 1, goff)
                    phase(b + 1, 0, goff)
                phase(nb - 1, 1, goff, do_prefetch=False)
                wait_scat(1)

            plsc.subcore_barrier()
            out_row = (core if nc <= 2 else gc) * N_PAD + r0
            pltpu.sync_copy(s_sh.at[pl.ds(r0, STRIPE)],
                            s_out.at[pl.ds(out_row, STRIPE)])

    return pl.kernel(body, mesh=_sc_mesh(), out_type=out_type,
                     scratch_types=scratch_types,
                     compiler_params=_sc_params())


def _sc_edge(g3, src_p, dst_p, dx, dy, wsp, z128):
    nc = g3.shape[0]
    fn = _make_sc_edge(nc)
    g2 = g3.reshape(nc * N_PAD, 128)
    out = fn(g2, src_p, dst_p, dx, dy, wsp, z128)
    return out[0].reshape(-1, N_PAD, 128)


def _make_sc_dxy():
    nb = E_PAD // 32 // 128

    out_type = [
        jax.ShapeDtypeStruct((E_PAD,), jnp.float32),
        jax.ShapeDtypeStruct((E_PAD,), jnp.float32),
    ]
    scratch_types = [
        pltpu.VMEM((128,), jnp.int32),
        pltpu.VMEM((128,), jnp.int32),
        pltpu.VMEM((128,), jnp.float32),
        pltpu.VMEM((128,), jnp.float32),
        pltpu.VMEM((N_PAD,), jnp.float32),
        pltpu.VMEM((N_PAD,), jnp.float32),
    ]

    def body(src_hbm, dst_hbm, posx_hbm, posy_hbm, dx_out, dy_out,
             idx_s, idx_d, dxb, dyb, posx_v, posy_v):
        core = lax.axis_index("c")
        sid = lax.axis_index("s")
        pltpu.sync_copy(posx_hbm, posx_v)
        pltpu.sync_copy(posy_hbm, posy_v)
        ebase = (core * 16 + sid) * (nb * 128)

        @pl.loop(0, nb)
        def _(b):
            off = ebase + b * 128
            pltpu.sync_copy(src_hbm.at[pl.ds(off, 128)], idx_s)
            pltpu.sync_copy(dst_hbm.at[pl.ds(off, 128)], idx_d)
            for j in range(8):
                sl = pl.ds(j * 16, 16)
                sv = idx_s[sl]
                dv = idx_d[sl]
                dxb[sl] = (plsc.load_gather(posx_v, [sv])
                           - plsc.load_gather(posx_v, [dv]))
                dyb[sl] = (plsc.load_gather(posy_v, [sv])
                           - plsc.load_gather(posy_v, [dv]))
            pltpu.sync_copy(dxb, dx_out.at[pl.ds(off, 128)])
            pltpu.sync_copy(dyb, dy_out.at[pl.ds(off, 128)])

    return pl.kernel(body, mesh=_sc_mesh(), out_type=out_type,
                     scratch_types=scratch_types,
                     compiler_params=_sc_params())


def _sc_dxy(src_p, dst_p, posx, posy):
    return _make_sc_dxy()(src_p, dst_p, posx, posy)


def kernel(h, pos, edge_index, params):
    p = params
    src = edge_index[0]
    dst = edge_index[1]
    src_p = jnp.pad(src, (0, E_PAD - E))
    dst_p = jnp.pad(dst, (0, E_PAD - E), constant_values=TRASH)

    x = jnp.pad(h, ((0, N_PAD - N), (0, 0)))
    pos_p = jnp.pad(pos, ((0, N_PAD - N), (0, 0)))
    posx = pos_p[:, 0]
    posy = pos_p[:, 1]

    z128 = jnp.zeros((N_SP // 16, 128), jnp.float32)

    dxe, dye = _sc_dxy(src_p, dst_p, posx, posy)
    ones_g = jnp.ones((1, N_PAD, 128), jnp.float32)
    wz = jnp.zeros((256,), jnp.float32)
    cnt_full = _sc_edge(ones_g, src_p, dst_p, dxe, dye, wz, z128)
    cnt2 = cnt_full[:, :, :16] * 0.0625
    for i in range(4):
        wa = p["W%d" % (2 * i)]
        ba = p["b%d" % (2 * i)]
        wb = p["W%d" % (2 * i + 1)]
        bb = p["b%d" % (2 * i + 1)]
        fi = wa.shape[0] - 2
        hdim = wa.shape[1]
        nc = hdim // 128
        wax = wa[:fi]
        wap = wa[fi:]
        wsp = jnp.transpose(wap.reshape(2, nc, 128), (1, 0, 2)).reshape(-1)
        g3 = _stage_a(x, wax, ba.reshape(1, -1))
        s3 = _sc_edge(g3, src_p, dst_p, dxe, dye, wsp, z128)
        if hdim == 128:
            wb3 = jnp.broadcast_to(wb[None], (2,) + wb.shape)
        else:
            wb3 = wb.reshape(hdim // 128, 128, wb.shape[1])
        x = _stage_c(s3, wb3, cnt2, bb.reshape(1, -1), True)

    x = _dense(x, p["W8"], p["b8"].reshape(1, -1), True)
    x = _dense(x, p["W9"], p["b9"].reshape(1, -1), True)
    w10 = jnp.pad(p["W10"], ((0, 0), (0, 128 - p["W10"].shape[1])))
    b10 = jnp.pad(p["b10"], (0, 128 - p["b10"].shape[0]))
    x = _dense(x, w10, b10.reshape(1, -1), False)
    return x[:N, :40]


# speedup vs baseline: 1.8089x; 1.8089x over previous
import dataclasses
import functools

import jax
import jax.numpy as jnp
from jax import lax
from jax.experimental import pallas as pl
from jax.experimental.pallas import tpu as pltpu
from jax.experimental.pallas import tpu_sc as plsc

N = 10000
E = 160000
N_PAD = 10240
N_SP = 10112
TRASH = 10000
E_PAD = 163840
BM = 1024
PREC = jax.lax.Precision.HIGHEST


def _stage_a_body(x_ref, wax_ref, ba_ref, g_ref):
    g_ref[0] = jnp.dot(x_ref[...], wax_ref[...], precision=PREC,
                       preferred_element_type=jnp.float32) + ba_ref[...]


def _stage_a(x, wax, ba):
    fi, h = wax.shape
    nc = h // 128
    nm = N_PAD // BM
    grid = (nm, nc)
    return pl.pallas_call(
        _stage_a_body,
        grid=grid,
        in_specs=[
            pl.BlockSpec((BM, fi), lambda m, o: (m, 0)),
            pl.BlockSpec((fi, 128), lambda m, o: (0, o)),
            pl.BlockSpec((1, 128), lambda m, o: (0, o)),
        ],
        out_specs=pl.BlockSpec((1, BM, 128), lambda m, o: (o, m, 0)),
        out_shape=jax.ShapeDtypeStruct((nc, N_PAD, 128), jnp.float32),
    )(x, wax, ba)


def _stage_c_body(nk, relu, s_ref, wb_ref, cnt_ref, bb_ref, o_ref, acc_ref):
    k = pl.program_id(2)

    @pl.when(k == 0)
    def _():
        acc_ref[...] = jnp.zeros_like(acc_ref)

    acc_ref[...] += jnp.dot(s_ref[0], wb_ref[0], precision=PREC,
                            preferred_element_type=jnp.float32)

    @pl.when(k == nk - 1)
    def _():
        csum = jnp.sum(cnt_ref[...], axis=(0, 2))[:, None]
        rc = 1.0 / jnp.maximum(csum, 1.0)
        ind = jnp.minimum(csum, 1.0)
        out = acc_ref[...] * rc + ind * bb_ref[...]
        if relu:
            out = jnp.maximum(out, 0.0)
        o_ref[...] = out


def _stage_c(s, wb3, cnt2, bb, relu):
    nk = wb3.shape[0]
    o = wb3.shape[2]
    bo = min(o, 256)
    nm = N_PAD // BM
    no = o // bo
    grid = (nm, no, nk)
    return pl.pallas_call(
        functools.partial(_stage_c_body, nk, relu),
        grid=grid,
        in_specs=[
            pl.BlockSpec((1, BM, 128), lambda m, o_, k: (k, m, 0)),
            pl.BlockSpec((1, 128, bo), lambda m, o_, k: (k, 0, o_)),
            pl.BlockSpec((2, BM, 16), lambda m, o_, k: (0, m, 0)),
            pl.BlockSpec((1, bo), lambda m, o_, k: (0, o_)),
        ],
        out_specs=pl.BlockSpec((BM, bo), lambda m, o_, k: (m, o_)),
        out_shape=jax.ShapeDtypeStruct((N_PAD, o), jnp.float32),
        scratch_shapes=[pltpu.VMEM((BM, bo), jnp.float32)],
    )(s, wb3, cnt2, bb)


def _dense_body(relu, x_ref, w_ref, b_ref, o_ref):
    out = jnp.dot(x_ref[...], w_ref[...], precision=PREC,
                  preferred_element_type=jnp.float32) + b_ref[...]
    if relu:
        out = jnp.maximum(out, 0.0)
    o_ref[...] = out


def _dense(x, w, b, relu):
    k, o = w.shape
    bo = min(o, 512)
    grid = (N_PAD // BM, o // bo)
    return pl.pallas_call(
        functools.partial(_dense_body, relu),
        grid=grid,
        in_specs=[
            pl.BlockSpec((BM, k), lambda m, o_: (m, 0)),
            pl.BlockSpec((k, bo), lambda m, o_: (0, o_)),
            pl.BlockSpec((1, bo), lambda m, o_: (0, o_)),
        ],
        out_specs=pl.BlockSpec((BM, bo), lambda m, o_: (m, o_)),
        out_shape=jax.ShapeDtypeStruct((N_PAD, o), jnp.float32),
    )(x, w, b)


STRIPE = N_SP // 16


def _sc_mesh():
    return plsc.VectorSubcoreMesh(core_axis_name="c", subcore_axis_name="s")


def _sc_params():
    cp = pltpu.CompilerParams()
    if "needs_layout_passes" in pltpu.CompilerParams.__dataclass_fields__:
        cp = dataclasses.replace(cp, needs_layout_passes=False)
    return cp


_SYNC = True


def _make_sc_edge(nc):
    edge_split = nc == 1
    passes = 1 if nc <= 2 else nc // 2
    n_out = 2 if nc == 1 else nc
    B = 64
    nb = 80 if edge_split else 160

    out_type = [jax.ShapeDtypeStruct((n_out * N_PAD, 128), jnp.float32)]

    scratch_types = (
        [pltpu.VMEM((B, 128), jnp.float32) for _ in range(2)]
        + [pltpu.VMEM((B,), jnp.int32) for _ in range(4)]
        + [pltpu.VMEM((B,), jnp.float32) for _ in range(4)]
        + [
            pltpu.VMEM((256,), jnp.float32),
            pltpu.VMEM_SHARED((N_SP, 128), jnp.float32),
        ]
        + [pltpu.SemaphoreType.DMA for _ in range(4)]
    )

    def body(g2, src_hbm, dst_hbm, dx_hbm, dy_hbm, wsp_hbm, z128, s_out,
             *scr):
        rows = scr[0:2]
        idd = scr[2:4]
        ids2 = scr[4:6]
        dxb = scr[6:8]
        dyb = scr[8:10]
        w_v, s_sh = scr[10:12]
        gsem = scr[12:14]
        ssem = scr[14:16]

        core = lax.axis_index("c")
        sid = lax.axis_index("s")
        r0 = sid * STRIPE

        egids = [lax.iota(jnp.int32, 16) + 16 * j for j in range(4)]

        if edge_split:
            ebase = core * (E_PAD // 2) + sid * (nb * B)
        else:
            ebase = sid * (nb * B)

        def prefetch(b, i, goff):
            off = ebase + b * B
            pltpu.sync_copy(src_hbm.at[pl.ds(off, B)], ids2[i])
            pltpu.sync_copy(dst_hbm.at[pl.ds(off, B)], idd[i])
            pltpu.sync_copy(dx_hbm.at[pl.ds(off, B)], dxb[i])
            pltpu.sync_copy(dy_hbm.at[pl.ds(off, B)], dyb[i])
            for j in range(4):
                sl = pl.ds(j * 16, 16)
                ids2[i][sl] = ids2[i][sl] + goff
            pltpu.async_copy(g2.at[ids2[i]], rows[i], gsem[i])

        def wait_gather(i):
            pltpu.make_async_copy(g2.at[pl.ds(0, B)], rows[i], gsem[i]).wait()

        def wait_scat(i):
            pltpu.make_async_copy(g2.at[pl.ds(0, B)], rows[i], ssem[i]).wait()

        def compute(i):
            @pl.loop(0, B)
            def _(e):
                esplat = jnp.full((16,), e, jnp.int32)
                dxv = plsc.load_gather(dxb[i], [esplat])
                dyv = plsc.load_gather(dyb[i], [esplat])
                for j in range(8):
                    sl = pl.ds(j * 16, 16)
                    g = rows[i][e, sl]
                    m = jnp.maximum(
                        g + dxv * w_v[sl]
                        + dyv * w_v[pl.ds(128 + j * 16, 16)], 0.0)
                    rows[i][e, sl] = m

        def phase(b, i, goff, first=False, do_prefetch=True):
            wait_gather(i)
            compute(i)
            pltpu.async_copy(rows[i], s_sh.at[idd[i]], ssem[i], add=True)
            if not first:
                wait_scat(1 - i)
            if do_prefetch:
                pf = b + 1
                if isinstance(pf, int):
                    if pf < nb:
                        prefetch(pf, 1 - i, goff)
                else:
                    @pl.when(pf < nb)
                    def _():
                        prefetch(pf, 1 - i, goff)

        for p in range(passes):
            gc = 0 if edge_split else core * passes + p
            goff = jnp.full((16,), gc * N_PAD, jnp.int32)
            pltpu.sync_copy(wsp_hbm.at[pl.ds(gc * 256, 256)], w_v)
            pltpu.sync_copy(z128, s_sh.at[pl.ds(r0, STRIPE)])
            plsc.subcore_barrier()

            if _SYNC:
                @pl.loop(0, nb)
                def _(b):
                    prefetch(b, 0, goff)
                    wait_gather(0)
                    compute(0)
                    pltpu.sync_copy(rows[0], s_sh.at[idd[0]], add=True)
            else:
                prefetch(0, 0, goff)
                phase(0, 0, goff, first=True)
                k = (nb - 2) // 2
                @pl.loop(1, 1 + 2 * k, step=2)
                def _(b):
                    phase(b, 1, goff)
                    phase(b + 1, 0, goff)
                phase(nb - 1, 1, goff, do_prefetch=False)
                wait_scat(1)

            plsc.subcore_barrier()
            out_row = (core if nc <= 2 else gc) * N_PAD + r0
            pltpu.sync_copy(s_sh.at[pl.ds(r0, STRIPE)],
                            s_out.at[pl.ds(out_row, STRIPE)])

    return pl.kernel(body, mesh=_sc_mesh(), out_type=out_type,
                     scratch_types=scratch_types,
                     compiler_params=_sc_params())


def _sc_edge(g3, src_p, dst_p, dx, dy, wsp, z128):
    nc = g3.shape[0]
    fn = _make_sc_edge(nc)
    g2 = g3.reshape(nc * N_PAD, 128)
    out = fn(g2, src_p, dst_p, dx, dy, wsp, z128)
    return out[0].reshape(-1, N_PAD, 128)


def _make_sc_dxy():
    nb = E_PAD // 32 // 128

    out_type = [
        jax.ShapeDtypeStruct((E_PAD,), jnp.float32),
        jax.ShapeDtypeStruct((E_PAD,), jnp.float32),
    ]
    scratch_types = [
        pltpu.VMEM((128,), jnp.int32),
        pltpu.VMEM((128,), jnp.int32),
        pltpu.VMEM((128,), jnp.float32),
        pltpu.VMEM((128,), jnp.float32),
        pltpu.VMEM((N_PAD,), jnp.float32),
        pltpu.VMEM((N_PAD,), jnp.float32),
    ]

    def body(src_hbm, dst_hbm, posx_hbm, posy_hbm, dx_out, dy_out,
             idx_s, idx_d, dxb, dyb, posx_v, posy_v):
        core = lax.axis_index("c")
        sid = lax.axis_index("s")
        pltpu.sync_copy(posx_hbm, posx_v)
        pltpu.sync_copy(posy_hbm, posy_v)
        ebase = (core * 16 + sid) * (nb * 128)

        @pl.loop(0, nb)
        def _(b):
            off = ebase + b * 128
            pltpu.sync_copy(src_hbm.at[pl.ds(off, 128)], idx_s)
            pltpu.sync_copy(dst_hbm.at[pl.ds(off, 128)], idx_d)
            for j in range(8):
                sl = pl.ds(j * 16, 16)
                sv = idx_s[sl]
                dv = idx_d[sl]
                dxb[sl] = (plsc.load_gather(posx_v, [sv])
                           - plsc.load_gather(posx_v, [dv]))
                dyb[sl] = (plsc.load_gather(posy_v, [sv])
                           - plsc.load_gather(posy_v, [dv]))
            pltpu.sync_copy(dxb, dx_out.at[pl.ds(off, 128)])
            pltpu.sync_copy(dyb, dy_out.at[pl.ds(off, 128)])

    return pl.kernel(body, mesh=_sc_mesh(), out_type=out_type,
                     scratch_types=scratch_types,
                     compiler_params=_sc_params())


def _sc_dxy(src_p, dst_p, posx, posy):
    return _make_sc_dxy()(src_p, dst_p, posx, posy)


def kernel(h, pos, edge_index, params):
    p = params
    src = edge_index[0]
    dst = edge_index[1]
    src_p = jnp.pad(src, (0, E_PAD - E))
    dst_p = jnp.pad(dst, (0, E_PAD - E), constant_values=TRASH)

    x = jnp.pad(h, ((0, N_PAD - N), (0, 0)))
    pos_p = jnp.pad(pos, ((0, N_PAD - N), (0, 0)))
    posx = pos_p[:, 0]
    posy = pos_p[:, 1]

    z128 = jnp.zeros((N_SP // 16, 128), jnp.float32)

    dxe, dye = _sc_dxy(src_p, dst_p, posx, posy)
    ones_g = jnp.ones((1, N_PAD, 128), jnp.float32)
    wz = jnp.zeros((256,), jnp.float32)
    cnt_full = _sc_edge(ones_g, src_p, dst_p, dxe, dye, wz, z128)
    cnt2 = cnt_full[:, :, :16] * 0.0625
    for i in range(4):
        wa = p["W%d" % (2 * i)]
        ba = p["b%d" % (2 * i)]
        wb = p["W%d" % (2 * i + 1)]
        bb = p["b%d" % (2 * i + 1)]
        fi = wa.shape[0] - 2
        hdim = wa.shape[1]
        nc = hdim // 128
        wax = wa[:fi]
        wap = wa[fi:]
        wsp = jnp.transpose(wap.reshape(2, nc, 128), (1, 0, 2)).reshape(-1)
        g3 = _stage_a(x, wax, ba.reshape(1, -1))
        s3 = _sc_edge(g3, src_p, dst_p, dxe, dye, wsp, z128)
        if hdim == 128:
            wb3 = jnp.broadcast_to(wb[None], (2,) + wb.shape)
        else:
            wb3 = wb.reshape(hdim // 128, 128, wb.shape[1])
        x = _stage_c(s3, wb3, cnt2, bb.reshape(1, -1), True)

    x = _dense(x, p["W8"], p["b8"].reshape(1, -1), True)
    x = _dense(x, p["W9"], p["b9"].reshape(1, -1), True)
    w10 = jnp.pad(p["W10"], ((0, 0), (0, 128 - p["W10"].shape[1])))
    b10 = jnp.pad(p["b10"], (0, 128 - p["b10"].shape[0]))
    x = _dense(x, w10, b10.reshape(1, -1), False)
    return x[:N, :40]


# speedup vs baseline: 1.8812x; 1.0400x over previous
import dataclasses
import functools

import jax
import jax.numpy as jnp
from jax import lax
from jax.experimental import pallas as pl
from jax.experimental.pallas import tpu as pltpu
from jax.experimental.pallas import tpu_sc as plsc

N = 10000
E = 160000
N_PAD = 10240
N_SP = 10112
TRASH = 10000
E_PAD = 163840
BM = 1024
PREC = jax.lax.Precision.HIGHEST


def _stage_a_body(x_ref, wax_ref, ba_ref, g_ref):
    g_ref[0] = jnp.dot(x_ref[...], wax_ref[...], precision=PREC,
                       preferred_element_type=jnp.float32) + ba_ref[...]


def _stage_a(x, wax, ba):
    fi, h = wax.shape
    nc = h // 128
    nm = N_PAD // BM
    grid = (nm, nc)
    return pl.pallas_call(
        _stage_a_body,
        grid=grid,
        in_specs=[
            pl.BlockSpec((BM, fi), lambda m, o: (m, 0)),
            pl.BlockSpec((fi, 128), lambda m, o: (0, o)),
            pl.BlockSpec((1, 128), lambda m, o: (0, o)),
        ],
        out_specs=pl.BlockSpec((1, BM, 128), lambda m, o: (o, m, 0)),
        out_shape=jax.ShapeDtypeStruct((nc, N_PAD, 128), jnp.float32),
    )(x, wax, ba)


def _stage_c_body(nk, relu, s_ref, wb_ref, cnt_ref, bb_ref, o_ref, acc_ref):
    k = pl.program_id(2)

    @pl.when(k == 0)
    def _():
        acc_ref[...] = jnp.zeros_like(acc_ref)

    acc_ref[...] += jnp.dot(s_ref[0], wb_ref[0], precision=PREC,
                            preferred_element_type=jnp.float32)

    @pl.when(k == nk - 1)
    def _():
        csum = jnp.sum(cnt_ref[...], axis=(0, 2))[:, None]
        rc = 1.0 / jnp.maximum(csum, 1.0)
        ind = jnp.minimum(csum, 1.0)
        out = acc_ref[...] * rc + ind * bb_ref[...]
        if relu:
            out = jnp.maximum(out, 0.0)
        o_ref[...] = out


def _stage_c(s, wb3, cnt2, bb, relu):
    nk = wb3.shape[0]
    o = wb3.shape[2]
    bo = min(o, 256)
    nm = N_PAD // BM
    no = o // bo
    grid = (nm, no, nk)
    return pl.pallas_call(
        functools.partial(_stage_c_body, nk, relu),
        grid=grid,
        in_specs=[
            pl.BlockSpec((1, BM, 128), lambda m, o_, k: (k, m, 0)),
            pl.BlockSpec((1, 128, bo), lambda m, o_, k: (k, 0, o_)),
            pl.BlockSpec((2, BM, 16), lambda m, o_, k: (0, m, 0)),
            pl.BlockSpec((1, bo), lambda m, o_, k: (0, o_)),
        ],
        out_specs=pl.BlockSpec((BM, bo), lambda m, o_, k: (m, o_)),
        out_shape=jax.ShapeDtypeStruct((N_PAD, o), jnp.float32),
        scratch_shapes=[pltpu.VMEM((BM, bo), jnp.float32)],
    )(s, wb3, cnt2, bb)


def _dense_body(relu, x_ref, w_ref, b_ref, o_ref):
    out = jnp.dot(x_ref[...], w_ref[...], precision=PREC,
                  preferred_element_type=jnp.float32) + b_ref[...]
    if relu:
        out = jnp.maximum(out, 0.0)
    o_ref[...] = out


def _dense(x, w, b, relu):
    k, o = w.shape
    bo = min(o, 512)
    grid = (N_PAD // BM, o // bo)
    return pl.pallas_call(
        functools.partial(_dense_body, relu),
        grid=grid,
        in_specs=[
            pl.BlockSpec((BM, k), lambda m, o_: (m, 0)),
            pl.BlockSpec((k, bo), lambda m, o_: (0, o_)),
            pl.BlockSpec((1, bo), lambda m, o_: (0, o_)),
        ],
        out_specs=pl.BlockSpec((BM, bo), lambda m, o_: (m, o_)),
        out_shape=jax.ShapeDtypeStruct((N_PAD, o), jnp.float32),
    )(x, w, b)


STRIPE = N_SP // 16


def _sc_mesh():
    return plsc.VectorSubcoreMesh(core_axis_name="c", subcore_axis_name="s")


def _sc_params():
    cp = pltpu.CompilerParams()
    if "needs_layout_passes" in pltpu.CompilerParams.__dataclass_fields__:
        cp = dataclasses.replace(cp, needs_layout_passes=False)
    return cp


_SYNC = False


def _make_sc_edge(nc):
    edge_split = nc == 1
    passes = 1 if nc <= 2 else nc // 2
    n_out = 2 if nc == 1 else nc
    B = 64
    nb = 80 if edge_split else 160

    out_type = [jax.ShapeDtypeStruct((n_out * N_PAD, 128), jnp.float32)]

    scratch_types = (
        [pltpu.VMEM((B, 128), jnp.float32) for _ in range(2)]
        + [pltpu.VMEM((B,), jnp.int32) for _ in range(4)]
        + [pltpu.VMEM((B,), jnp.float32) for _ in range(4)]
        + [
            pltpu.VMEM((256,), jnp.float32),
            pltpu.VMEM_SHARED((N_SP, 128), jnp.float32),
        ]
        + [pltpu.SemaphoreType.DMA for _ in range(4)]
    )

    def body(g2, src_hbm, dst_hbm, dx_hbm, dy_hbm, wsp_hbm, z128, s_out,
             *scr):
        rows = scr[0:2]
        idd = scr[2:4]
        ids2 = scr[4:6]
        dxb = scr[6:8]
        dyb = scr[8:10]
        w_v, s_sh = scr[10:12]
        gsem = scr[12:14]
        ssem = scr[14:16]

        core = lax.axis_index("c")
        sid = lax.axis_index("s")
        r0 = sid * STRIPE

        egids = [lax.iota(jnp.int32, 16) + 16 * j for j in range(4)]

        if edge_split:
            ebase = core * (E_PAD // 2) + sid * (nb * B)
        else:
            ebase = sid * (nb * B)

        def prefetch(b, i, goff):
            off = ebase + b * B
            pltpu.sync_copy(src_hbm.at[pl.ds(off, B)], ids2[i])
            pltpu.sync_copy(dst_hbm.at[pl.ds(off, B)], idd[i])
            pltpu.sync_copy(dx_hbm.at[pl.ds(off, B)], dxb[i])
            pltpu.sync_copy(dy_hbm.at[pl.ds(off, B)], dyb[i])
            for j in range(4):
                sl = pl.ds(j * 16, 16)
                ids2[i][sl] = ids2[i][sl] + goff
            pltpu.async_copy(g2.at[ids2[i]], rows[i], gsem[i])

        def wait_gather(i):
            pltpu.make_async_copy(g2.at[pl.ds(0, B)], rows[i], gsem[i]).wait()

        def wait_scat(i):
            pltpu.make_async_copy(g2.at[pl.ds(0, B)], rows[i], ssem[i]).wait()

        def compute(i):
            @pl.loop(0, B)
            def _(e):
                esplat = jnp.full((16,), e, jnp.int32)
                dxv = plsc.load_gather(dxb[i], [esplat])
                dyv = plsc.load_gather(dyb[i], [esplat])
                for j in range(8):
                    sl = pl.ds(j * 16, 16)
                    g = rows[i][e, sl]
                    m = jnp.maximum(
                        g + dxv * w_v[sl]
                        + dyv * w_v[pl.ds(128 + j * 16, 16)], 0.0)
                    rows[i][e, sl] = m

        def phase(b, i, goff, first=False, do_prefetch=True):
            wait_gather(i)
            compute(i)
            pltpu.async_copy(rows[i], s_sh.at[idd[i]], ssem[i], add=True)
            if not first:
                wait_scat(1 - i)
            if do_prefetch:
                pf = b + 1
                if isinstance(pf, int):
                    if pf < nb:
                        prefetch(pf, 1 - i, goff)
                else:
                    @pl.when(pf < nb)
                    def _():
                        prefetch(pf, 1 - i, goff)

        for p in range(passes):
            gc = 0 if edge_split else core * passes + p
            goff = jnp.full((16,), gc * N_PAD, jnp.int32)
            pltpu.sync_copy(wsp_hbm.at[pl.ds(gc * 256, 256)], w_v)
            pltpu.sync_copy(z128, s_sh.at[pl.ds(r0, STRIPE)])
            plsc.subcore_barrier()

            if _SYNC:
                @pl.loop(0, nb)
                def _(b):
                    prefetch(b, 0, goff)
                    wait_gather(0)
                    compute(0)
                    pltpu.sync_copy(rows[0], s_sh.at[idd[0]], add=True)
            else:
                prefetch(0, 0, goff)
                phase(0, 0, goff, first=True)
                k = (nb - 2) // 2
                @pl.loop(1, 1 + 2 * k, step=2)
                def _(b):
                    phase(b, 1, goff)
                    phase(b + 1, 0, goff)
                phase(nb - 1, 1, goff, do_prefetch=False)
                wait_scat(1)

            plsc.subcore_barrier()
            out_row = (core if nc <= 2 else gc) * N_PAD + r0
            pltpu.sync_copy(s_sh.at[pl.ds(r0, STRIPE)],
                            s_out.at[pl.ds(out_row, STRIPE)])

    return pl.kernel(body, mesh=_sc_mesh(), out_type=out_type,
                     scratch_types=scratch_types,
                     compiler_params=_sc_params())


def _sc_edge(g3, src_p, dst_p, dx, dy, wsp, z128):
    nc = g3.shape[0]
    fn = _make_sc_edge(nc)
    g2 = g3.reshape(nc * N_PAD, 128)
    out = fn(g2, src_p, dst_p, dx, dy, wsp, z128)
    return out[0].reshape(-1, N_PAD, 128)


def _make_sc_dxy():
    nb = E_PAD // 32 // 128

    out_type = [
        jax.ShapeDtypeStruct((E_PAD,), jnp.float32),
        jax.ShapeDtypeStruct((E_PAD,), jnp.float32),
    ]
    scratch_types = [
        pltpu.VMEM((128,), jnp.int32),
        pltpu.VMEM((128,), jnp.int32),
        pltpu.VMEM((128,), jnp.float32),
        pltpu.VMEM((128,), jnp.float32),
        pltpu.VMEM((N_PAD,), jnp.float32),
        pltpu.VMEM((N_PAD,), jnp.float32),
    ]

    def body(src_hbm, dst_hbm, posx_hbm, posy_hbm, dx_out, dy_out,
             idx_s, idx_d, dxb, dyb, posx_v, posy_v):
        core = lax.axis_index("c")
        sid = lax.axis_index("s")
        pltpu.sync_copy(posx_hbm, posx_v)
        pltpu.sync_copy(posy_hbm, posy_v)
        ebase = (core * 16 + sid) * (nb * 128)

        @pl.loop(0, nb)
        def _(b):
            off = ebase + b * 128
            pltpu.sync_copy(src_hbm.at[pl.ds(off, 128)], idx_s)
            pltpu.sync_copy(dst_hbm.at[pl.ds(off, 128)], idx_d)
            for j in range(8):
                sl = pl.ds(j * 16, 16)
                sv = idx_s[sl]
                dv = idx_d[sl]
                dxb[sl] = (plsc.load_gather(posx_v, [sv])
                           - plsc.load_gather(posx_v, [dv]))
                dyb[sl] = (plsc.load_gather(posy_v, [sv])
                           - plsc.load_gather(posy_v, [dv]))
            pltpu.sync_copy(dxb, dx_out.at[pl.ds(off, 128)])
            pltpu.sync_copy(dyb, dy_out.at[pl.ds(off, 128)])

    return pl.kernel(body, mesh=_sc_mesh(), out_type=out_type,
                     scratch_types=scratch_types,
                     compiler_params=_sc_params())


def _sc_dxy(src_p, dst_p, posx, posy):
    return _make_sc_dxy()(src_p, dst_p, posx, posy)


def kernel(h, pos, edge_index, params):
    p = params
    src = edge_index[0]
    dst = edge_index[1]
    src_p = jnp.pad(src, (0, E_PAD - E))
    dst_p = jnp.pad(dst, (0, E_PAD - E), constant_values=TRASH)

    x = jnp.pad(h, ((0, N_PAD - N), (0, 0)))
    pos_p = jnp.pad(pos, ((0, N_PAD - N), (0, 0)))
    posx = pos_p[:, 0]
    posy = pos_p[:, 1]

    z128 = jnp.zeros((N_SP // 16, 128), jnp.float32)

    dxe, dye = _sc_dxy(src_p, dst_p, posx, posy)
    ones_g = jnp.ones((1, N_PAD, 128), jnp.float32)
    wz = jnp.zeros((256,), jnp.float32)
    cnt_full = _sc_edge(ones_g, src_p, dst_p, dxe, dye, wz, z128)
    cnt2 = cnt_full[:, :, :16] * 0.0625
    for i in range(4):
        wa = p["W%d" % (2 * i)]
        ba = p["b%d" % (2 * i)]
        wb = p["W%d" % (2 * i + 1)]
        bb = p["b%d" % (2 * i + 1)]
        fi = wa.shape[0] - 2
        hdim = wa.shape[1]
        nc = hdim // 128
        wax = wa[:fi]
        wap = wa[fi:]
        wsp = jnp.transpose(wap.reshape(2, nc, 128), (1, 0, 2)).reshape(-1)
        g3 = _stage_a(x, wax, ba.reshape(1, -1))
        s3 = _sc_edge(g3, src_p, dst_p, dxe, dye, wsp, z128)
        if hdim == 128:
            wb3 = jnp.broadcast_to(wb[None], (2,) + wb.shape)
        else:
            wb3 = wb.reshape(hdim // 128, 128, wb.shape[1])
        x = _stage_c(s3, wb3, cnt2, bb.reshape(1, -1), True)

    x = _dense(x, p["W8"], p["b8"].reshape(1, -1), True)
    x = _dense(x, p["W9"], p["b9"].reshape(1, -1), True)
    w10 = jnp.pad(p["W10"], ((0, 0), (0, 128 - p["W10"].shape[1])))
    b10 = jnp.pad(p["b10"], (0, 128 - p["b10"].shape[0]))
    x = _dense(x, w10, b10.reshape(1, -1), False)
    return x[:N, :40]


# speedup vs baseline: 2.8651x; 1.5230x over previous
import dataclasses
import functools

import jax
import jax.numpy as jnp
from jax import lax
from jax.experimental import pallas as pl
from jax.experimental.pallas import tpu as pltpu
from jax.experimental.pallas import tpu_sc as plsc

N = 10000
E = 160000
N_PAD = 10240
N_SP = 10112
TRASH = 10000
E_PAD = 163840
BM = 1024
PREC = jax.lax.Precision.HIGHEST


def _stage_a_body(x_ref, wax_ref, ba_ref, g_ref):
    g_ref[0] = jnp.dot(x_ref[...], wax_ref[...], precision=PREC,
                       preferred_element_type=jnp.float32) + ba_ref[...]


def _stage_a(x, wax, ba):
    fi, h = wax.shape
    nc = h // 128
    nm = N_PAD // BM
    grid = (nm, nc)
    return pl.pallas_call(
        _stage_a_body,
        grid=grid,
        in_specs=[
            pl.BlockSpec((BM, fi), lambda m, o: (m, 0)),
            pl.BlockSpec((fi, 128), lambda m, o: (0, o)),
            pl.BlockSpec((1, 128), lambda m, o: (0, o)),
        ],
        out_specs=pl.BlockSpec((1, BM, 128), lambda m, o: (o, m, 0)),
        out_shape=jax.ShapeDtypeStruct((nc, N_PAD, 128), jnp.float32),
    )(x, wax, ba)


def _stage_c_body(nk, relu, s_ref, wb_ref, cnt_ref, bb_ref, o_ref, acc_ref):
    k = pl.program_id(2)

    @pl.when(k == 0)
    def _():
        acc_ref[...] = jnp.zeros_like(acc_ref)

    acc_ref[...] += jnp.dot(s_ref[0], wb_ref[0], precision=PREC,
                            preferred_element_type=jnp.float32)

    @pl.when(k == nk - 1)
    def _():
        csum = jnp.sum(cnt_ref[...], axis=(0, 2))[:, None]
        rc = 1.0 / jnp.maximum(csum, 1.0)
        ind = jnp.minimum(csum, 1.0)
        out = acc_ref[...] * rc + ind * bb_ref[...]
        if relu:
            out = jnp.maximum(out, 0.0)
        o_ref[...] = out


def _stage_c(s, wb3, cnt2, bb, relu):
    nk = wb3.shape[0]
    o = wb3.shape[2]
    bo = min(o, 256)
    nm = N_PAD // BM
    no = o // bo
    grid = (nm, no, nk)
    return pl.pallas_call(
        functools.partial(_stage_c_body, nk, relu),
        grid=grid,
        in_specs=[
            pl.BlockSpec((1, BM, 128), lambda m, o_, k: (k, m, 0)),
            pl.BlockSpec((1, 128, bo), lambda m, o_, k: (k, 0, o_)),
            pl.BlockSpec((2, BM, 16), lambda m, o_, k: (0, m, 0)),
            pl.BlockSpec((1, bo), lambda m, o_, k: (0, o_)),
        ],
        out_specs=pl.BlockSpec((BM, bo), lambda m, o_, k: (m, o_)),
        out_shape=jax.ShapeDtypeStruct((N_PAD, o), jnp.float32),
        scratch_shapes=[pltpu.VMEM((BM, bo), jnp.float32)],
    )(s, wb3, cnt2, bb)


def _dense_body(relu, x_ref, w_ref, b_ref, o_ref):
    out = jnp.dot(x_ref[...], w_ref[...], precision=PREC,
                  preferred_element_type=jnp.float32) + b_ref[...]
    if relu:
        out = jnp.maximum(out, 0.0)
    o_ref[...] = out


def _dense(x, w, b, relu):
    k, o = w.shape
    bo = min(o, 512)
    grid = (N_PAD // BM, o // bo)
    return pl.pallas_call(
        functools.partial(_dense_body, relu),
        grid=grid,
        in_specs=[
            pl.BlockSpec((BM, k), lambda m, o_: (m, 0)),
            pl.BlockSpec((k, bo), lambda m, o_: (0, o_)),
            pl.BlockSpec((1, bo), lambda m, o_: (0, o_)),
        ],
        out_specs=pl.BlockSpec((BM, bo), lambda m, o_: (m, o_)),
        out_shape=jax.ShapeDtypeStruct((N_PAD, o), jnp.float32),
    )(x, w, b)


STRIPE = N_SP // 16


def _sc_mesh():
    return plsc.VectorSubcoreMesh(core_axis_name="c", subcore_axis_name="s")


def _sc_params():
    cp = pltpu.CompilerParams()
    if "needs_layout_passes" in pltpu.CompilerParams.__dataclass_fields__:
        cp = dataclasses.replace(cp, needs_layout_passes=False)
    return cp


_SYNC = False


def _make_sc_edge(nc):
    edge_split = nc == 1
    passes = 1 if nc <= 2 else nc // 2
    n_out = 2 if nc == 1 else nc
    B = 64
    nb = 80 if edge_split else 160

    out_type = [jax.ShapeDtypeStruct((n_out * N_PAD, 128), jnp.float32)]

    scratch_types = (
        [pltpu.VMEM((B, 128), jnp.float32) for _ in range(2)]
        + [pltpu.VMEM((B,), jnp.int32) for _ in range(4)]
        + [pltpu.VMEM((B,), jnp.float32) for _ in range(4)]
        + [
            pltpu.VMEM((256,), jnp.float32),
            pltpu.VMEM_SHARED((N_SP, 128), jnp.float32),
        ]
        + [pltpu.SemaphoreType.DMA for _ in range(4)]
    )

    def body(g2, src_hbm, dst_hbm, dx_hbm, dy_hbm, wsp_hbm, z128, s_out,
             *scr):
        rows = scr[0:2]
        idd = scr[2:4]
        ids2 = scr[4:6]
        dxb = scr[6:8]
        dyb = scr[8:10]
        w_v, s_sh = scr[10:12]
        gsem = scr[12:14]
        ssem = scr[14:16]

        core = lax.axis_index("c")
        sid = lax.axis_index("s")
        r0 = sid * STRIPE

        egids = [lax.iota(jnp.int32, 16) + 16 * j for j in range(4)]

        if edge_split:
            ebase = core * (E_PAD // 2) + sid * (nb * B)
        else:
            ebase = sid * (nb * B)

        def prefetch(b, i, goff):
            off = ebase + b * B
            pltpu.sync_copy(src_hbm.at[pl.ds(off, B)], ids2[i])
            pltpu.sync_copy(dst_hbm.at[pl.ds(off, B)], idd[i])
            pltpu.sync_copy(dx_hbm.at[pl.ds(off, B)], dxb[i])
            pltpu.sync_copy(dy_hbm.at[pl.ds(off, B)], dyb[i])
            for j in range(4):
                sl = pl.ds(j * 16, 16)
                ids2[i][sl] = ids2[i][sl] + goff
            pltpu.async_copy(g2.at[ids2[i]], rows[i], gsem[i])

        def wait_gather(i):
            pltpu.make_async_copy(g2.at[pl.ds(0, B)], rows[i], gsem[i]).wait()

        def wait_scat(i):
            pltpu.make_async_copy(g2.at[pl.ds(0, B)], rows[i], ssem[i]).wait()

        def compute(i):
            w0j = [w_v[pl.ds(j * 16, 16)] for j in range(8)]
            w1j = [w_v[pl.ds(128 + j * 16, 16)] for j in range(8)]

            @pl.loop(0, B)
            def _(e):
                esplat = jnp.full((16,), e, jnp.int32)
                dxv = plsc.load_gather(dxb[i], [esplat])
                dyv = plsc.load_gather(dyb[i], [esplat])
                for j in range(8):
                    sl = pl.ds(j * 16, 16)
                    g = rows[i][e, sl]
                    m = jnp.maximum(g + dxv * w0j[j] + dyv * w1j[j], 0.0)
                    rows[i][e, sl] = m

        def phase(b, i, goff, first=False, do_prefetch=True):
            wait_gather(i)
            compute(i)
            pltpu.async_copy(rows[i], s_sh.at[idd[i]], ssem[i], add=True)
            if not first:
                wait_scat(1 - i)
            if do_prefetch:
                pf = b + 1
                if isinstance(pf, int):
                    if pf < nb:
                        prefetch(pf, 1 - i, goff)
                else:
                    @pl.when(pf < nb)
                    def _():
                        prefetch(pf, 1 - i, goff)

        for p in range(passes):
            gc = 0 if edge_split else core * passes + p
            goff = jnp.full((16,), gc * N_PAD, jnp.int32)
            pltpu.sync_copy(wsp_hbm.at[pl.ds(gc * 256, 256)], w_v)
            pltpu.sync_copy(z128, s_sh.at[pl.ds(r0, STRIPE)])
            plsc.subcore_barrier()

            if _SYNC:
                @pl.loop(0, nb)
                def _(b):
                    prefetch(b, 0, goff)
                    wait_gather(0)
                    compute(0)
                    pltpu.sync_copy(rows[0], s_sh.at[idd[0]], add=True)
            else:
                prefetch(0, 0, goff)
                phase(0, 0, goff, first=True)
                k = (nb - 2) // 2
                @pl.loop(1, 1 + 2 * k, step=2)
                def _(b):
                    phase(b, 1, goff)
                    phase(b + 1, 0, goff)
                phase(nb - 1, 1, goff, do_prefetch=False)
                wait_scat(1)

            plsc.subcore_barrier()
            out_row = (core if nc <= 2 else gc) * N_PAD + r0
            pltpu.sync_copy(s_sh.at[pl.ds(r0, STRIPE)],
                            s_out.at[pl.ds(out_row, STRIPE)])

    return pl.kernel(body, mesh=_sc_mesh(), out_type=out_type,
                     scratch_types=scratch_types,
                     compiler_params=_sc_params())


def _sc_edge(g3, src_p, dst_p, dx, dy, wsp, z128):
    nc = g3.shape[0]
    fn = _make_sc_edge(nc)
    g2 = g3.reshape(nc * N_PAD, 128)
    out = fn(g2, src_p, dst_p, dx, dy, wsp, z128)
    return out[0].reshape(-1, N_PAD, 128)


def _make_sc_dxy():
    nb = E_PAD // 32 // 128

    out_type = [
        jax.ShapeDtypeStruct((E_PAD,), jnp.float32),
        jax.ShapeDtypeStruct((E_PAD,), jnp.float32),
    ]
    scratch_types = [
        pltpu.VMEM((128,), jnp.int32),
        pltpu.VMEM((128,), jnp.int32),
        pltpu.VMEM((128,), jnp.float32),
        pltpu.VMEM((128,), jnp.float32),
        pltpu.VMEM((N_PAD,), jnp.float32),
        pltpu.VMEM((N_PAD,), jnp.float32),
    ]

    def body(src_hbm, dst_hbm, posx_hbm, posy_hbm, dx_out, dy_out,
             idx_s, idx_d, dxb, dyb, posx_v, posy_v):
        core = lax.axis_index("c")
        sid = lax.axis_index("s")
        pltpu.sync_copy(posx_hbm, posx_v)
        pltpu.sync_copy(posy_hbm, posy_v)
        ebase = (core * 16 + sid) * (nb * 128)

        @pl.loop(0, nb)
        def _(b):
            off = ebase + b * 128
            pltpu.sync_copy(src_hbm.at[pl.ds(off, 128)], idx_s)
            pltpu.sync_copy(dst_hbm.at[pl.ds(off, 128)], idx_d)
            for j in range(8):
                sl = pl.ds(j * 16, 16)
                sv = idx_s[sl]
                dv = idx_d[sl]
                dxb[sl] = (plsc.load_gather(posx_v, [sv])
                           - plsc.load_gather(posx_v, [dv]))
                dyb[sl] = (plsc.load_gather(posy_v, [sv])
                           - plsc.load_gather(posy_v, [dv]))
            pltpu.sync_copy(dxb, dx_out.at[pl.ds(off, 128)])
            pltpu.sync_copy(dyb, dy_out.at[pl.ds(off, 128)])

    return pl.kernel(body, mesh=_sc_mesh(), out_type=out_type,
                     scratch_types=scratch_types,
                     compiler_params=_sc_params())


def _sc_dxy(src_p, dst_p, posx, posy):
    return _make_sc_dxy()(src_p, dst_p, posx, posy)


def kernel(h, pos, edge_index, params):
    p = params
    src = edge_index[0]
    dst = edge_index[1]
    src_p = jnp.pad(src, (0, E_PAD - E))
    dst_p = jnp.pad(dst, (0, E_PAD - E), constant_values=TRASH)

    x = jnp.pad(h, ((0, N_PAD - N), (0, 0)))
    pos_p = jnp.pad(pos, ((0, N_PAD - N), (0, 0)))
    posx = pos_p[:, 0]
    posy = pos_p[:, 1]

    z128 = jnp.zeros((N_SP // 16, 128), jnp.float32)

    dxe, dye = _sc_dxy(src_p, dst_p, posx, posy)
    ones_g = jnp.ones((1, N_PAD, 128), jnp.float32)
    wz = jnp.zeros((256,), jnp.float32)
    cnt_full = _sc_edge(ones_g, src_p, dst_p, dxe, dye, wz, z128)
    cnt2 = cnt_full[:, :, :16] * 0.0625
    for i in range(4):
        wa = p["W%d" % (2 * i)]
        ba = p["b%d" % (2 * i)]
        wb = p["W%d" % (2 * i + 1)]
        bb = p["b%d" % (2 * i + 1)]
        fi = wa.shape[0] - 2
        hdim = wa.shape[1]
        nc = hdim // 128
        wax = wa[:fi]
        wap = wa[fi:]
        wsp = jnp.transpose(wap.reshape(2, nc, 128), (1, 0, 2)).reshape(-1)
        g3 = _stage_a(x, wax, ba.reshape(1, -1))
        s3 = _sc_edge(g3, src_p, dst_p, dxe, dye, wsp, z128)
        if hdim == 128:
            wb3 = jnp.broadcast_to(wb[None], (2,) + wb.shape)
        else:
            wb3 = wb.reshape(hdim // 128, 128, wb.shape[1])
        x = _stage_c(s3, wb3, cnt2, bb.reshape(1, -1), True)

    x = _dense(x, p["W8"], p["b8"].reshape(1, -1), True)
    x = _dense(x, p["W9"], p["b9"].reshape(1, -1), True)
    w10 = jnp.pad(p["W10"], ((0, 0), (0, 128 - p["W10"].shape[1])))
    b10 = jnp.pad(p["b10"], (0, 128 - p["b10"].shape[0]))
    x = _dense(x, w10, b10.reshape(1, -1), False)
    return x[:N, :40]
